# baseline (speedup 1.0000x reference)
import jax
import jax.numpy as jnp
from jax import lax
from jax.experimental import pallas as pl
from jax.experimental.pallas import tpu as pltpu
from jax.experimental.pallas import tpu_sc as plsc

B = 16384
C = 5
E = 64
NW = 32
CHUNK = 128
NCHUNK = B // (NW * CHUNK)
L = 16


def _body(tgt_hbm, ctx_hbm, ttab_hbm, ctab_hbm, out_hbm,
          tidx, cidx, wrows, crows, outv, sem):
    wid = lax.axis_index("s") * 2 + lax.axis_index("c")
    iota = lax.iota(jnp.int32, L)
    lane15 = iota == (L - 1)
    perms = [iota ^ sh for sh in (8, 4, 2, 1)]

    dnums = lax.GatherDimensionNumbers(
        offset_dims=(), collapsed_slice_dims=(0,), start_index_map=(0,))

    def shuffle(v, p):
        return lax.gather(v, p[:, None], dnums, slice_sizes=(1,),
                          mode=lax.GatherScatterMode.PROMISE_IN_BOUNDS)

    def hsum(v):
        for p in perms:
            v = v + shuffle(v, p)
        return v

    for k in range(NCHUNK):
        ci = wid * NCHUNK + k

        pltpu.sync_copy(tgt_hbm.at[pl.ds(ci * CHUNK, CHUNK)], tidx)
        pltpu.sync_copy(ctx_hbm.at[pl.ds(ci * CHUNK * C, CHUNK * C)], cidx)

        cps = [pltpu.async_copy(ttab_hbm.at[tidx], wrows, sem)]
        for j in range(C):
            cps.append(pltpu.async_copy(
                ctab_hbm.at[cidx.at[pl.ds(j * CHUNK, CHUNK)]],
                crows.at[pl.ds(j * CHUNK, CHUNK)], sem))
        for cp in cps:
            cp.wait()

        def row_step(b, carry):
            w = [wrows[b, pl.ds(v * L, L)] for v in range(4)]
            comb = None
            for c in range(C):
                r = b * C + c
                p = w[0] * crows[r, pl.ds(0, L)]
                for v in range(1, 4):
                    p = p + w[v] * crows[r, pl.ds(v * L, L)]
                s = hsum(p)
                comb = s if comb is None else jnp.where(iota == c, s, comb)
            outv[pl.ds(b * C, L)] = comb
            return carry

        lax.fori_loop(0, CHUNK, row_step, 0)
        pltpu.sync_copy(outv.at[pl.ds(0, CHUNK * C)],
                        out_hbm.at[pl.ds(ci * CHUNK * C, CHUNK * C)])


@jax.jit
def _run(tgt2d, ctxflat2d, target_table, context_table):
    mesh = plsc.VectorSubcoreMesh(core_axis_name="c", subcore_axis_name="s")
    return pl.kernel(
        _body,
        out_type=jax.ShapeDtypeStruct((B * C,), jnp.float32),
        mesh=mesh,
        compiler_params=pltpu.CompilerParams(use_tc_tiling_on_sc=False),
        scratch_types=[
            pltpu.VMEM((CHUNK,), jnp.int32),
            pltpu.VMEM((CHUNK * C,), jnp.int32),
            pltpu.VMEM((CHUNK, E), jnp.float32),
            pltpu.VMEM((CHUNK * C, E), jnp.float32),
            pltpu.VMEM((CHUNK * C + L,), jnp.float32),
            pltpu.SemaphoreType.DMA,
        ],
    )(tgt2d, ctxflat2d, target_table, context_table)


def kernel(target, context, target_table, context_table):
    tgt = target.reshape(B).astype(jnp.int32)
    ctxflat = context.reshape(B * C).astype(jnp.int32)
    out = _run(tgt, ctxflat, target_table, context_table)
    return out.reshape(B, C)


# speedup vs baseline: 1.0542x; 1.0542x over previous
import jax
import jax.numpy as jnp
from jax import lax
from jax.experimental import pallas as pl
from jax.experimental.pallas import tpu as pltpu
from jax.experimental.pallas import tpu_sc as plsc

B = 16384
C = 5
E = 64
EP = 128
NW = 32
CHUNK = 128
NCHUNK = B // (NW * CHUNK)
L = 16


def _body(tgt_hbm, ctx_hbm, ttab_hbm, ctab_hbm, out_hbm,
          tidx, cidx, wrows, crows, outv, sem):
    wid = lax.axis_index("s") * 2 + lax.axis_index("c")
    iota = lax.iota(jnp.int32, L)
    perms = [iota ^ sh for sh in (8, 4, 2, 1)]
    dnums = lax.GatherDimensionNumbers(
        offset_dims=(), collapsed_slice_dims=(0,), start_index_map=(0,))

    def hsum(v):
        for p in perms:
            v = v + lax.gather(v, p[:, None], dnums, slice_sizes=(1,),
                               mode=lax.GatherScatterMode.PROMISE_IN_BOUNDS)
        return v

    for k in range(NCHUNK):
        pltpu.sync_copy(tgt_hbm.at[wid, 0, pl.ds(k * CHUNK, CHUNK)], tidx)
        pltpu.sync_copy(ctx_hbm.at[wid, 0, pl.ds(k * CHUNK * C, CHUNK * C)],
                        cidx)

        cps = [pltpu.async_copy(ttab_hbm.at[tidx], wrows, sem)]
        for j in range(C):
            cps.append(pltpu.async_copy(
                ctab_hbm.at[cidx.at[pl.ds(j * CHUNK, CHUNK)]],
                crows.at[pl.ds(j * CHUNK, CHUNK)], sem))
        for cp in cps:
            cp.wait()

        def row_step(b, carry):
            w = [wrows[b, pl.ds(v * L, L)] for v in range(4)]
            comb = None
            for c in range(C):
                r = b * C + c
                p = w[0] * crows[r, pl.ds(0, L)]
                for v in range(1, 4):
                    p = p + w[v] * crows[r, pl.ds(v * L, L)]
                s = hsum(p)
                comb = s if comb is None else jnp.where(iota == c, s, comb)
            outv[pl.ds(b * C, L)] = comb
            return carry

        lax.fori_loop(0, CHUNK, row_step, 0)
        pltpu.sync_copy(outv.at[pl.ds(0, CHUNK * C)],
                        out_hbm.at[wid, 0, pl.ds(k * CHUNK * C, CHUNK * C)])


@jax.jit
def _run(tgt3, ctx3, ttab_p, ctab_p):
    mesh = plsc.VectorSubcoreMesh(core_axis_name="c", subcore_axis_name="s")
    return pl.kernel(
        _body,
        out_type=jax.ShapeDtypeStruct((NW, 1, B * C // NW), jnp.float32),
        mesh=mesh,
        scratch_types=[
            pltpu.VMEM((CHUNK,), jnp.int32),
            pltpu.VMEM((CHUNK * C,), jnp.int32),
            pltpu.VMEM((CHUNK, EP), jnp.float32),
            pltpu.VMEM((CHUNK * C, EP), jnp.float32),
            pltpu.VMEM((CHUNK * C + L,), jnp.float32),
            pltpu.SemaphoreType.DMA,
        ],
    )(tgt3, ctx3, ttab_p, ctab_p)


def kernel(target, context, target_table, context_table):
    tgt3 = target.reshape(NW, 1, B // NW).astype(jnp.int32)
    ctx3 = context.reshape(NW, 1, B * C // NW).astype(jnp.int32)
    ttab_p = jnp.pad(target_table, ((0, 0), (0, EP - E)))
    ctab_p = jnp.pad(context_table, ((0, 0), (0, EP - E)))
    out = _run(tgt3, ctx3, ttab_p, ctab_p)
    return out.reshape(B, C)
